# baseline (speedup 1.0000x reference)
import functools

import jax
import jax.numpy as jnp
from jax import lax
from jax.experimental import pallas as pl
from jax.experimental.pallas import tpu as pltpu
from jax.experimental.pallas import tpu_sc as plsc

_TS = 524288
_GRID = [16, 22, 30, 42, 58, 80, 110, 152, 210, 290, 400, 552, 762, 1052, 1453, 2006]
_N = 262144
_NLEV = 16
_F = 4
_NC = 2
_NS = 16
_NW = _NC * _NS
_PPW = _N // _NW
_C = 32
_NCHUNK = _PPW // _C
_G = _C // 16
_H1 = 19349663
_H2 = 83492791
_MASK = _TS - 1

_RPL = 32
_BL = 32
_ROWS_PER_BATCH = _RPL * _BL

_NROWS = [min(g**3, _TS) for g in _GRID]
_BASES = []
_NBATCH = []
_REM = []
_tot = 0
for _v in _NROWS:
    _BASES.append(_tot)
    _nb = _v // _ROWS_PER_BATCH
    _rem = _v - _nb * _ROWS_PER_BATCH
    _NBATCH.append(_nb)
    _REM.append(_rem)
    _tot += _nb * _BL + (_BL if _rem else 0)
_TOTLINES = _tot


_REMLEVS = [l for l in range(_NLEV) if _REM[l]]


def _encode_body(xf, *rest):
    tables = rest[:_NLEV]
    remt = rest[_NLEV]
    out, pack = rest[_NLEV + 1 : _NLEV + 3]
    xb, idxb, lob, cfb, rows, outb, vbuf, lines, sem = rest[_NLEV + 3 :]

    cid = lax.axis_index("c")
    sid = lax.axis_index("s")
    wid = sid * _NC + cid
    iota = lax.iota(jnp.int32, 16)

    for lev in range(_NLEV):
        nb = _NBATCH[lev]
        base = _BASES[lev]
        cnt = jnp.maximum(0, (nb - sid + _NS - 1) // _NS)

        def rp_batch(k, c, lev=lev, base=base):
            b = sid + k * _NS
            pltpu.sync_copy(
                tables[lev].at[:, pl.ds(b * _ROWS_PER_BATCH, _ROWS_PER_BATCH)],
                vbuf,
            )

            def line_body(j, c2):
                for h in range(8):
                    fidx = j * 128 + h * 16 + iota
                    val = plsc.load_gather(vbuf, [fidx & 3, fidx >> 2])
                    lines[j, pl.ds(h * 16, 16)] = val
                return c2

            lax.fori_loop(0, _BL, line_body, 0, unroll=False)
            pltpu.sync_copy(lines, pack.at[pl.ds(base + b * _BL, _BL), :])
            return c

        lax.fori_loop(0, cnt, rp_batch, 0, unroll=False)

        rem = _REM[lev]
        if rem:
            nrl = -(-rem // _RPL)

            li = _REMLEVS.index(lev)

            @pl.when(sid == 0)
            def _(lev=lev, base=base, nb=nb, rem=rem, nrl=nrl, li=li):
                pltpu.sync_copy(
                    remt.at[:, pl.ds(li * 1024, 1024)],
                    vbuf.at[:, pl.ds(0, 1024)],
                )
                def rem_line(j, c2):
                    for h in range(8):
                        fidx = j * 128 + h * 16 + iota
                        fidx = jnp.minimum(fidx, rem * _F - 1)
                        val = plsc.load_gather(vbuf, [fidx & 3, fidx >> 2])
                        lines[j, pl.ds(h * 16, 16)] = val
                    return c2

                lax.fori_loop(0, nrl, rem_line, 0, unroll=False)
                nrl8 = -(-nrl // 8) * 8
                pltpu.sync_copy(
                    lines.at[pl.ds(0, nrl8), :],
                    pack.at[pl.ds(base + nb * _BL, nrl8), :],
                )

    plsc.subcore_barrier()

    def chunk_body(ci, carry):
        pbase = wid * _PPW + ci * _C
        for d in range(3):
            pltpu.sync_copy(xf.at[pl.ds(d * _N + pbase, _C)], xb.at[d])

        for lev in range(_NLEV):
            gs = _GRID[lev]
            hashed = gs**3 > _TS
            lbase = _BASES[lev]

            def a_body(g, c, gs=gs, hashed=hashed, lbase=lbase):
                p0 = g * 16
                px = xb[0, pl.ds(p0, 16)]
                py = xb[1, pl.ds(p0, 16)]
                pz = xb[2, pl.ds(p0, 16)]

                def axis_prep(p):
                    frac = jnp.minimum(jnp.maximum(0.5 * p + 0.5, 0.0), 1.0)
                    fi = 0.5 + float(gs - 2) * frac
                    ui = fi.astype(jnp.int32)
                    ui = jnp.minimum(ui, gs - 2)
                    cf = fi - ui.astype(jnp.float32)
                    cf = cf * cf * (3.0 - 2.0 * cf)
                    return ui, cf

                ix, cfx = axis_prep(px)
                iy, cfy = axis_prep(py)
                iz, cfz = axis_prep(pz)
                cfb[0, pl.ds(p0, 16)] = cfx
                cfb[1, pl.ds(p0, 16)] = cfy
                cfb[2, pl.ds(p0, 16)] = cfz

                if hashed:
                    hy0 = iy * _H1
                    hy1 = hy0 + _H1
                    hz0 = iz * _H2
                    hz1 = hz0 + _H2
                    ix1 = ix + 1
                    parts = []
                    for hx in (ix, ix1):
                        for hy in (hy0, hy1):
                            for hz in (hz0, hz1):
                                parts.append((hx ^ hy ^ hz) & _MASK)
                else:
                    ty0 = iy * gs
                    ty1 = ty0 + gs
                    tz0 = iz * (gs * gs)
                    tz1 = tz0 + gs * gs
                    ix1 = ix + 1
                    parts = []
                    for tx in (ix, ix1):
                        for ty in (ty0, ty1):
                            for tz in (tz0, tz1):
                                parts.append(tx + ty + tz)
                for corner, idx in enumerate(parts):
                    col = (corner & 1) * _C + p0
                    idxb[corner >> 1, pl.ds(col, 16)] = lbase + (idx >> 5)
                    lob[corner, pl.ds(p0, 16)] = (idx & 31) * _F
                return c

            lax.fori_loop(0, _G, a_body, 0, unroll=False)

            descs = [
                pltpu.async_copy(
                    pack.at[idxb.at[q]],
                    rows.at[pl.ds(q * 2 * _C, 2 * _C)],
                    sem,
                )
                for q in range(4)
            ]
            for d in descs:
                d.wait()

            def b_body(g, c, lev=lev):
                p0 = g * 16
                cfx = cfb[0, pl.ds(p0, 16)]
                cfy = cfb[1, pl.ds(p0, 16)]
                cfz = cfb[2, pl.ds(p0, 16)]
                wx = (1.0 - cfx, cfx)
                wy = (1.0 - cfy, cfy)
                wz = (1.0 - cfz, cfz)
                pvec = p0 + iota
                accs = [jnp.zeros((16,), jnp.float32) for _ in range(_F)]
                corner = 0
                for ox in (0, 1):
                    for oy in (0, 1):
                        wxy = wx[ox] * wy[oy]
                        for oz in (0, 1):
                            w = wxy * wz[oz]
                            lo = lob[corner, pl.ds(p0, 16)]
                            rrow = corner * _C + pvec
                            for f in range(_F):
                                feat = plsc.load_gather(rows, [rrow, lo + f])
                                accs[f] = accs[f] + w * feat
                            corner += 1
                for f in range(_F):
                    col = jnp.full((16,), _F * lev + f, jnp.int32)
                    plsc.store_scatter(outb, [pvec, col], accs[f])
                return c

            lax.fori_loop(0, _G, b_body, 0, unroll=False)

        pltpu.sync_copy(outb, out.at[pl.ds(pbase, _C), :])
        return carry

    lax.fori_loop(0, _NCHUNK, chunk_body, 0, unroll=False)


@jax.jit
def kernel(x, tables):
    xf = x.T.reshape(-1)
    segs = []
    for l in _REMLEVS:
        seg = tables[l][_NBATCH[l] * _ROWS_PER_BATCH :]
        segs.append(jnp.pad(seg, ((0, 1024 - seg.shape[0]), (0, 0))))
    remt = jnp.concatenate(segs, axis=0).T
    mesh = plsc.VectorSubcoreMesh(core_axis_name="c", subcore_axis_name="s")
    fn = functools.partial(
        pl.kernel,
        out_type=(
            jax.ShapeDtypeStruct((_N, _NLEV * _F), jnp.float32),
            jax.ShapeDtypeStruct((_TOTLINES, 128), jnp.float32),
        ),
        mesh=mesh,
        scratch_types=[
            pltpu.VMEM((3, _C), jnp.float32),
            pltpu.VMEM((4, 2 * _C), jnp.int32),
            pltpu.VMEM((8, _C), jnp.int32),
            pltpu.VMEM((3, _C), jnp.float32),
            pltpu.VMEM((8 * _C, 128), jnp.float32),
            pltpu.VMEM((_C, _NLEV * _F), jnp.float32),
            pltpu.VMEM((_F, _ROWS_PER_BATCH), jnp.float32),
            pltpu.VMEM((_BL, 128), jnp.float32),
            pltpu.SemaphoreType.DMA,
        ],
        compiler_params=pltpu.CompilerParams(needs_layout_passes=False),
    )(_encode_body)
    out, _ = fn(xf, *(t.T for t in tables), remt)
    return out
